# baseline (speedup 1.0000x reference)
import functools

import jax
import jax.numpy as jnp
from jax import lax
from jax.experimental import pallas as pl
from jax.experimental.pallas import tpu as pltpu
from jax.experimental.pallas import tpu_sc as plsc

N_NODES = 10000
D_FEAT = 128
D_ATTR = 4
AVG_NEIGH = 10.0

NC = 2
NS = 16
NW = NC * NS
CHUNK = 128
N_PAD = 10240
ROWS_PER_SUB = N_PAD // NS

H = 4
BE = 5120

NB_G = 6
NB_S = 2


def _dot_t(lhs_t, rhs):
    return jax.lax.dot_general(
        lhs_t, rhs, (((0,), (0,)), ((), ())),
        preferred_element_type=jnp.float32)


def _edge_body(ef_ref, at_ref, nfj_ref, wup_ref, w1_ref, w2_ref, w3_ref,
               w4_ref, wl_ref, rep_ref, y_ref):
    h = jax.nn.silu(_dot_t(ef_ref[...], w1_ref[...]))
    h = jax.nn.silu(jnp.dot(h, w2_ref[...],
                            preferred_element_type=jnp.float32))
    h = jax.nn.silu(jnp.dot(h, w3_ref[...],
                            preferred_element_type=jnp.float32))
    tpw = jnp.dot(h.astype(jnp.bfloat16), w4_ref[...],
                  preferred_element_type=jnp.float32)
    tpw = tpw * _dot_t(at_ref[...], rep_ref[...])
    xj = jnp.dot(nfj_ref[...].astype(jnp.bfloat16), wup_ref[...],
                 preferred_element_type=jnp.float32)
    acc = jnp.zeros((BE, D_FEAT), jnp.float32)
    for v in range(D_ATTR):
        zv = xj * tpw[:, v * D_FEAT:(v + 1) * D_FEAT]
        acc = acc + jnp.dot(zv.astype(jnp.bfloat16), wl_ref[v],
                            preferred_element_type=jnp.float32)
    y_ref[...] = acc


def _sum_body(p0_ref, p1_ref, p2_ref, p3_ref, o_ref):
    acc = p0_ref[0] + p0_ref[1]
    for r in (p1_ref, p2_ref, p3_ref):
        acc = acc + r[0] + r[1]
    o_ref[...] = acc


def _gather_body(k, x_hbm, idx_hbm, out_hbm, idx_v, bufs, sem_g, sem_w):
    c = lax.axis_index("c")
    s = lax.axis_index("s")
    wid = s * NC + c
    pltpu.sync_copy(idx_hbm.at[wid], idx_v)

    gd = [None] * NB_G
    wd = [None] * NB_G
    for j in range(k + 1):
        if j >= 1:
            p = j - 1
            pb = p % NB_G
            gd[pb].wait()
            wd[pb] = pltpu.async_copy(
                bufs.at[pb],
                out_hbm.at[pl.ds((p * NW + wid) * CHUNK, CHUNK)],
                sem_w.at[pb])
        if j < k:
            b = j % NB_G
            if j >= NB_G:
                wd[b].wait()
            gd[b] = pltpu.async_copy(x_hbm.at[idx_v.at[j]], bufs.at[b],
                                     sem_g.at[b])
    for b in range(NB_G):
        if wd[b] is not None:
            wd[b].wait()


def _scatter_body(k, y_hbm, idx_hbm, out_hbm, acc_sh, idx_v, bufs,
                  sem_l, sem_s):
    c = lax.axis_index("c")
    s = lax.axis_index("s")
    wid = s * NC + c

    z = jnp.zeros((16,), jnp.float32)

    def zfill(r, carry):
        for i in range(D_FEAT // 16):
            bufs[0, r, pl.ds(16 * i, 16)] = z
        return carry

    lax.fori_loop(0, CHUNK, zfill, 0)
    base = s * ROWS_PER_SUB
    full = ROWS_PER_SUB // CHUNK
    for t in range(full):
        pltpu.sync_copy(bufs.at[0], acc_sh.at[pl.ds(base + t * CHUNK, CHUNK)])
    plsc.subcore_barrier()

    pltpu.sync_copy(idx_hbm.at[wid], idx_v)
    ld = [None] * NB_S
    sd = [None] * NB_S
    for j in range(k + 1):
        if j >= 1:
            p = j - 1
            pb = p % NB_S
            ld[pb].wait()
            sd[pb] = pltpu.async_copy(bufs.at[pb], acc_sh.at[idx_v.at[p]],
                                      sem_s.at[pb], add=True)
        if j < k:
            b = j % NB_S
            if j >= NB_S:
                sd[b].wait()
            ld[b] = pltpu.async_copy(
                y_hbm.at[pl.ds((j * NW + wid) * CHUNK, CHUNK)], bufs.at[b],
                sem_l.at[b])
    for b in range(NB_S):
        if sd[b] is not None:
            sd[b].wait()
    plsc.subcore_barrier()

    obase = c * N_PAD + base
    for t in range(full):
        pltpu.sync_copy(acc_sh.at[pl.ds(base + t * CHUNK, CHUNK)],
                        out_hbm.at[pl.ds(obase + t * CHUNK, CHUNK)])


def kernel(node_feats, edge_index, edge_attrs, edge_feats,
           W_up, W1, W2, W3, W4, W_lin):
    n_nodes = node_feats.shape[0]
    n_edges = edge_feats.shape[0]
    d_edge = edge_feats.shape[1]
    hidden = W1.shape[1]
    wnum = D_FEAT * D_ATTR

    part = -(-n_edges // (H * NW * CHUNK)) * NW * CHUNK
    ep = H * part
    pad = ep - n_edges
    k_p = part // (NW * CHUNK)
    blocks_p = part // BE
    sender = jnp.pad(edge_index[0].astype(jnp.int32), (0, pad))
    receiver = jnp.pad(edge_index[1].astype(jnp.int32), (0, pad),
                       constant_values=n_nodes)
    ef_t = jnp.pad(edge_feats.T, ((0, 0), (0, pad)))
    at_t = jnp.pad(edge_attrs.T, ((0, 0), (0, pad)))
    idx_s = sender.reshape(H, k_p, NW, CHUNK).transpose(0, 2, 1, 3)
    idx_r = receiver.reshape(H, k_p, NW, CHUNK).transpose(0, 2, 1, 3)

    perm = (D_ATTR * jnp.arange(D_FEAT)[None, :]
            + jnp.arange(D_ATTR)[:, None]).reshape(-1)
    W4p = W4[:, perm].astype(jnp.bfloat16)
    rep = (jnp.arange(wnum)[None, :] // D_FEAT
           == jnp.arange(D_ATTR)[:, None]).astype(jnp.float32)
    Wlp = (W_lin[perm, :] / AVG_NEIGH).reshape(
        D_ATTR, D_FEAT, D_FEAT).astype(jnp.bfloat16)

    W_upb = W_up.astype(jnp.bfloat16)

    mesh = plsc.VectorSubcoreMesh(core_axis_name="c", subcore_axis_name="s")
    gather_call = pl.kernel(
        functools.partial(_gather_body, k_p),
        out_type=jax.ShapeDtypeStruct((part, D_FEAT), jnp.float32),
        mesh=mesh,
        scratch_types=[
            pltpu.VMEM((k_p, CHUNK), jnp.int32),
            pltpu.VMEM((NB_G, CHUNK, D_FEAT), jnp.float32),
            pltpu.SemaphoreType.DMA((NB_G,)),
            pltpu.SemaphoreType.DMA((NB_G,)),
        ],
    )
    scatter_call = pl.kernel(
        functools.partial(_scatter_body, k_p),
        out_type=jax.ShapeDtypeStruct((NC * N_PAD, D_FEAT), jnp.float32),
        mesh=mesh,
        scratch_types=[
            pltpu.VMEM_SHARED((N_PAD, D_FEAT), jnp.float32),
            pltpu.VMEM((k_p, CHUNK), jnp.int32),
            pltpu.VMEM((NB_S, CHUNK, D_FEAT), jnp.float32),
            pltpu.SemaphoreType.DMA((NB_S,)),
            pltpu.SemaphoreType.DMA((NB_S,)),
        ],
    )

    partials = []
    for p in range(H):
        x_j = gather_call(node_feats, idx_s[p])

        def _col_map(i, p=p):
            return (0, p * blocks_p + i)

        y = pl.pallas_call(
            _edge_body,
            grid=(blocks_p,),
            in_specs=[
                pl.BlockSpec((d_edge, BE), _col_map),
                pl.BlockSpec((D_ATTR, BE), _col_map),
                pl.BlockSpec((BE, D_FEAT), lambda i: (i, 0)),
                pl.BlockSpec((D_FEAT, D_FEAT), lambda i: (0, 0)),
                pl.BlockSpec((d_edge, hidden), lambda i: (0, 0)),
                pl.BlockSpec((hidden, hidden), lambda i: (0, 0)),
                pl.BlockSpec((hidden, hidden), lambda i: (0, 0)),
                pl.BlockSpec((hidden, wnum), lambda i: (0, 0)),
                pl.BlockSpec((D_ATTR, D_FEAT, D_FEAT), lambda i: (0, 0, 0)),
                pl.BlockSpec((D_ATTR, wnum), lambda i: (0, 0)),
            ],
            out_specs=pl.BlockSpec((BE, D_FEAT), lambda i: (i, 0)),
            out_shape=jax.ShapeDtypeStruct((part, D_FEAT), jnp.float32),
        )(ef_t, at_t, x_j, W_upb, W1, W2, W3, W4p, Wlp, rep)

        partials.append(scatter_call(y, idx_r[p]))

    BN = 2048
    pspec = pl.BlockSpec((NC, BN, D_FEAT), lambda i: (0, i, 0))
    out = pl.pallas_call(
        _sum_body,
        grid=(N_PAD // BN,),
        in_specs=[pspec] * H,
        out_specs=pl.BlockSpec((BN, D_FEAT), lambda i: (i, 0)),
        out_shape=jax.ShapeDtypeStruct((N_PAD, D_FEAT), jnp.float32),
    )(*[q.reshape(NC, N_PAD, D_FEAT) for q in partials])

    return out[:n_nodes]


# speedup vs baseline: 1.1626x; 1.1626x over previous
import functools

import jax
import jax.numpy as jnp
from jax import lax
from jax.experimental import pallas as pl
from jax.experimental.pallas import tpu as pltpu
from jax.experimental.pallas import tpu_sc as plsc

N_NODES = 10000
D_FEAT = 128
D_ATTR = 4
AVG_NEIGH = 10.0

NC = 2
NS = 16
NW = NC * NS
CHUNK = 128
N_PAD = 10240
ROWS_PER_SUB = N_PAD // NS

H = 4
BE = 4096

NB_G = 6
NB_S = 2


def _dot_t(lhs_t, rhs):
    return jax.lax.dot_general(
        lhs_t, rhs, (((0,), (0,)), ((), ())),
        preferred_element_type=jnp.float32)


def _up_body(nf_ref, w_ref, x_ref):
    x_ref[...] = jnp.dot(nf_ref[...], w_ref[...],
                         preferred_element_type=jnp.float32)


def _edge_body(ef_ref, at_ref, xj_ref, w1_ref, w2_ref, w3_ref,
               w4_ref, wl_ref, rep_ref, y_ref):
    h = jax.nn.silu(_dot_t(ef_ref[...], w1_ref[...]))
    h = jax.nn.silu(jnp.dot(h, w2_ref[...],
                            preferred_element_type=jnp.float32))
    h = jax.nn.silu(jnp.dot(h, w3_ref[...],
                            preferred_element_type=jnp.float32))
    tpw = jnp.dot(h.astype(jnp.bfloat16), w4_ref[...],
                  preferred_element_type=jnp.float32)
    tpw = tpw * _dot_t(at_ref[...], rep_ref[...])
    xj = xj_ref[...]
    acc = jnp.zeros((BE, D_FEAT), jnp.float32)
    for v in range(D_ATTR):
        zv = xj * tpw[:, v * D_FEAT:(v + 1) * D_FEAT]
        acc = acc + jnp.dot(zv.astype(jnp.bfloat16), wl_ref[v],
                            preferred_element_type=jnp.float32)
    y_ref[...] = acc


def _sum_body(p0_ref, p1_ref, p2_ref, p3_ref, o_ref):
    acc = p0_ref[0] + p0_ref[1]
    for r in (p1_ref, p2_ref, p3_ref):
        acc = acc + r[0] + r[1]
    o_ref[...] = acc


def _gather_body(k, x_hbm, idx_hbm, out_hbm, idx_v, bufs, sem_g, sem_w):
    c = lax.axis_index("c")
    s = lax.axis_index("s")
    wid = s * NC + c
    pltpu.sync_copy(idx_hbm.at[wid], idx_v)

    gd = [None] * NB_G
    wd = [None] * NB_G
    for j in range(k + 1):
        if j >= 1:
            p = j - 1
            pb = p % NB_G
            gd[pb].wait()
            wd[pb] = pltpu.async_copy(
                bufs.at[pb],
                out_hbm.at[pl.ds((p * NW + wid) * CHUNK, CHUNK)],
                sem_w.at[pb])
        if j < k:
            b = j % NB_G
            if j >= NB_G:
                wd[b].wait()
            gd[b] = pltpu.async_copy(x_hbm.at[idx_v.at[j]], bufs.at[b],
                                     sem_g.at[b])
    for b in range(NB_G):
        if wd[b] is not None:
            wd[b].wait()


def _scatter_body(k, y_hbm, idx_hbm, out_hbm, acc_sh, idx_v, bufs,
                  sem_l, sem_s):
    c = lax.axis_index("c")
    s = lax.axis_index("s")
    wid = s * NC + c

    z = jnp.zeros((16,), jnp.float32)

    def zfill(r, carry):
        for i in range(D_FEAT // 16):
            bufs[0, r, pl.ds(16 * i, 16)] = z
        return carry

    lax.fori_loop(0, CHUNK, zfill, 0)
    base = s * ROWS_PER_SUB
    full = ROWS_PER_SUB // CHUNK
    for t in range(full):
        pltpu.sync_copy(bufs.at[0], acc_sh.at[pl.ds(base + t * CHUNK, CHUNK)])
    plsc.subcore_barrier()

    pltpu.sync_copy(idx_hbm.at[wid], idx_v)
    ld = [None] * NB_S
    sd = [None] * NB_S
    for j in range(k + 1):
        if j >= 1:
            p = j - 1
            pb = p % NB_S
            ld[pb].wait()
            sd[pb] = pltpu.async_copy(bufs.at[pb], acc_sh.at[idx_v.at[p]],
                                      sem_s.at[pb], add=True)
        if j < k:
            b = j % NB_S
            if j >= NB_S:
                sd[b].wait()
            ld[b] = pltpu.async_copy(
                y_hbm.at[pl.ds((j * NW + wid) * CHUNK, CHUNK)], bufs.at[b],
                sem_l.at[b])
    for b in range(NB_S):
        if sd[b] is not None:
            sd[b].wait()
    plsc.subcore_barrier()

    obase = c * N_PAD + base
    for t in range(full):
        pltpu.sync_copy(acc_sh.at[pl.ds(base + t * CHUNK, CHUNK)],
                        out_hbm.at[pl.ds(obase + t * CHUNK, CHUNK)])


def kernel(node_feats, edge_index, edge_attrs, edge_feats,
           W_up, W1, W2, W3, W4, W_lin):
    n_nodes = node_feats.shape[0]
    n_edges = edge_feats.shape[0]
    d_edge = edge_feats.shape[1]
    hidden = W1.shape[1]
    wnum = D_FEAT * D_ATTR

    ep = -(-n_edges // (NW * CHUNK)) * NW * CHUNK
    pad = ep - n_edges
    tch = ep // CHUNK
    q = tch // NW
    q0 = max(1, round(q * 0.05))
    q1 = max(1, round(q * 0.15))
    q2 = (q - q0 - q1 + 1) // 2
    q3 = q - q0 - q1 - q2
    sizes = [q0 * NW, q1 * NW, q2 * NW, q3 * NW]
    starts = [sum(sizes[:p]) for p in range(H)]
    sender = jnp.pad(edge_index[0].astype(jnp.int32), (0, pad))
    receiver = jnp.pad(edge_index[1].astype(jnp.int32), (0, pad),
                       constant_values=n_nodes)
    ef_t = jnp.pad(edge_feats.T, ((0, 0), (0, pad)))
    at_t = jnp.pad(edge_attrs.T, ((0, 0), (0, pad)))
    snd2 = sender.reshape(tch, CHUNK)
    rcv2 = receiver.reshape(tch, CHUNK)

    perm = (D_ATTR * jnp.arange(D_FEAT)[None, :]
            + jnp.arange(D_ATTR)[:, None]).reshape(-1)
    W4p = W4[:, perm].astype(jnp.bfloat16)
    rep = (jnp.arange(wnum)[None, :] // D_FEAT
           == jnp.arange(D_ATTR)[:, None]).astype(jnp.float32)
    Wlp = (W_lin[perm, :] / AVG_NEIGH).reshape(
        D_ATTR, D_FEAT, D_FEAT).astype(jnp.bfloat16)

    x = pl.pallas_call(
        _up_body,
        out_shape=jax.ShapeDtypeStruct((n_nodes, D_FEAT), jnp.float32),
    )(node_feats, W_up)

    mesh = plsc.VectorSubcoreMesh(core_axis_name="c", subcore_axis_name="s")
    gather_calls, scatter_calls = {}, {}

    def get_calls(k_p, n_rows):
        if k_p not in gather_calls:
            gather_calls[k_p] = pl.kernel(
                functools.partial(_gather_body, k_p),
                out_type=jax.ShapeDtypeStruct((n_rows, D_FEAT), jnp.float32),
                mesh=mesh,
                scratch_types=[
                    pltpu.VMEM((k_p, CHUNK), jnp.int32),
                    pltpu.VMEM((NB_G, CHUNK, D_FEAT), jnp.float32),
                    pltpu.SemaphoreType.DMA((NB_G,)),
                    pltpu.SemaphoreType.DMA((NB_G,)),
                ],
            )
            scatter_calls[k_p] = pl.kernel(
                functools.partial(_scatter_body, k_p),
                out_type=jax.ShapeDtypeStruct((NC * N_PAD, D_FEAT),
                                              jnp.float32),
                mesh=mesh,
                scratch_types=[
                    pltpu.VMEM_SHARED((N_PAD, D_FEAT), jnp.float32),
                    pltpu.VMEM((k_p, CHUNK), jnp.int32),
                    pltpu.VMEM((NB_S, CHUNK, D_FEAT), jnp.float32),
                    pltpu.SemaphoreType.DMA((NB_S,)),
                    pltpu.SemaphoreType.DMA((NB_S,)),
                ],
            )
        return gather_calls[k_p], scatter_calls[k_p]

    partials = []
    for p in range(H):
        sz = sizes[p]
        k_p = sz // NW
        n_rows = sz * CHUNK
        blocks_p = n_rows // BE
        col0 = starts[p] * CHUNK // BE
        gather_call, scatter_call = get_calls(k_p, n_rows)
        idx_s = snd2[starts[p]:starts[p] + sz].reshape(
            k_p, NW, CHUNK).transpose(1, 0, 2)
        idx_r = rcv2[starts[p]:starts[p] + sz].reshape(
            k_p, NW, CHUNK).transpose(1, 0, 2)

        x_j = gather_call(x, idx_s)

        def _col_map(i, col0=col0):
            return (0, col0 + i)

        y = pl.pallas_call(
            _edge_body,
            grid=(blocks_p,),
            in_specs=[
                pl.BlockSpec((d_edge, BE), _col_map),
                pl.BlockSpec((D_ATTR, BE), _col_map),
                pl.BlockSpec((BE, D_FEAT), lambda i: (i, 0)),
                pl.BlockSpec((d_edge, hidden), lambda i: (0, 0)),
                pl.BlockSpec((hidden, hidden), lambda i: (0, 0)),
                pl.BlockSpec((hidden, hidden), lambda i: (0, 0)),
                pl.BlockSpec((hidden, wnum), lambda i: (0, 0)),
                pl.BlockSpec((D_ATTR, D_FEAT, D_FEAT), lambda i: (0, 0, 0)),
                pl.BlockSpec((D_ATTR, wnum), lambda i: (0, 0)),
            ],
            out_specs=pl.BlockSpec((BE, D_FEAT), lambda i: (i, 0)),
            out_shape=jax.ShapeDtypeStruct((n_rows, D_FEAT), jnp.float32),
        )(ef_t, at_t, x_j, W1, W2, W3, W4p, Wlp, rep)

        partials.append(scatter_call(y, idx_r))

    BN = 2048
    pspec = pl.BlockSpec((NC, BN, D_FEAT), lambda i: (0, i, 0))
    out = pl.pallas_call(
        _sum_body,
        grid=(N_PAD // BN,),
        in_specs=[pspec] * H,
        out_specs=pl.BlockSpec((BN, D_FEAT), lambda i: (i, 0)),
        out_shape=jax.ShapeDtypeStruct((N_PAD, D_FEAT), jnp.float32),
    )(*[qq.reshape(NC, N_PAD, D_FEAT) for qq in partials])

    return out[:n_nodes]
